# baseline (speedup 1.0000x reference)
import functools

import jax
import jax.numpy as jnp
from jax import lax
from jax.experimental import pallas as pl
from jax.experimental.pallas import tpu as pltpu
from jax.experimental.pallas import tpu_sc as plsc

EMB = 64
NN = 10000
NP = 10240
NE = 320000
EP = 327680
NWORK = 32
EPW = EP // NWORK
CH = 80
NCHUNK = EPW // CH
ZROWS = NP // 16
BR = 1024
GRID = NP // BR
EPS = 1e-5


def _lane_gather(x, idx):
    dn = lax.GatherDimensionNumbers(
        offset_dims=(), collapsed_slice_dims=(0,), start_index_map=(0,))
    return lax.gather(x, idx[:, None], dn, (1,),
                      mode=lax.GatherScatterMode.PROMISE_IN_BOUNDS)


def _lane_sum(x):
    idx = lax.iota(jnp.int32, 16)
    for sh in (8, 4, 2, 1):
        x = x + _lane_gather(x, idx ^ sh)
    return x


def _rsqrt_nr(v):
    i = lax.bitcast_convert_type(v, jnp.int32)
    i = jnp.int32(0x5F3759DF) - lax.shift_right_logical(i, 1)
    y = lax.bitcast_convert_type(i, jnp.float32)
    for _ in range(3):
        y = y * (1.5 - 0.5 * v * y * y)
    return y


def _build_sc_conv():
    mesh = plsc.VectorSubcoreMesh(core_axis_name="c", subcore_axis_name="s")

    NB = 4
    NQ = NCHUNK // NB
    bufset = [pltpu.VMEM((CH,), jnp.int32), pltpu.VMEM((CH,), jnp.int32),
              pltpu.VMEM((2 * CH,), jnp.float32),
              pltpu.VMEM((CH, EMB), jnp.float32),
              pltpu.VMEM((CH, EMB), jnp.float32),
              pltpu.VMEM((CH, EMB), jnp.float32),
              pltpu.SemaphoreType.DMA, pltpu.SemaphoreType.DMA,
              pltpu.SemaphoreType.DMA]

    @functools.partial(
        pl.kernel,
        mesh=mesh,
        out_type=jax.ShapeDtypeStruct((2 * NP, EMB), jnp.float32),
        compiler_params=pltpu.CompilerParams(use_tc_tiling_on_sc=False),
        scratch_types=(bufset * NB) + [
            pltpu.VMEM((4 * EMB,), jnp.float32),
            pltpu.VMEM_SHARED((NP, EMB), jnp.float32),
        ],
    )
    def sc_conv(a_h, b_h, src_h, dst_h, ea_h, w_h, out_h, *scr):
        wv, acc = scr[-2], scr[-1]
        srcv = tuple(scr[9 * b + 0] for b in range(NB))
        dstv = tuple(scr[9 * b + 1] for b in range(NB))
        eav = tuple(scr[9 * b + 2] for b in range(NB))
        arows = tuple(scr[9 * b + 3] for b in range(NB))
        brows = tuple(scr[9 * b + 4] for b in range(NB))
        trows = tuple(scr[9 * b + 5] for b in range(NB))
        si = tuple(scr[9 * b + 6] for b in range(NB))
        sg = tuple(scr[9 * b + 7] for b in range(NB))
        ss = tuple(scr[9 * b + 8] for b in range(NB))
        trows0 = trows[0]
        cid = lax.axis_index("c")
        sid = lax.axis_index("s")
        wid = cid * 16 + sid

        def _zrow(r, carry):
            zero = jnp.zeros((16,), jnp.float32)
            for t in range(4):
                trows0[r, pl.ds(16 * t, 16)] = zero
            return carry
        lax.fori_loop(0, CH, _zrow, 0)
        base_r = sid * ZROWS
        for k in range(ZROWS // CH):
            pltpu.sync_copy(trows0, acc.at[pl.ds(base_r + k * CH, CH)])
        plsc.subcore_barrier()

        pltpu.sync_copy(w_h, wv)
        we0 = [wv[pl.ds(16 * t, 16)] for t in range(4)]
        we1 = [wv[pl.ds(EMB + 16 * t, 16)] for t in range(4)]
        gv = [wv[pl.ds(2 * EMB + 16 * t, 16)] for t in range(4)]
        bv = [wv[pl.ds(3 * EMB + 16 * t, 16)] for t in range(4)]
        pidx = lax.iota(jnp.int32, 16) ^ 1

        ebase = wid * EPW

        def issue_idx(k, b):
            off = ebase + k * CH
            pltpu.async_copy(src_h.at[pl.ds(off, CH)], srcv[b], si[b])
            pltpu.async_copy(dst_h.at[pl.ds(off, CH)], dstv[b], si[b])
            pltpu.async_copy(ea_h.at[pl.ds(2 * off, 2 * CH)], eav[b], si[b])

        def wait_idx(b):
            pltpu.make_async_copy(src_h.at[pl.ds(0, CH)], srcv[b],
                                  si[b]).wait()
            pltpu.make_async_copy(dst_h.at[pl.ds(0, CH)], dstv[b],
                                  si[b]).wait()
            pltpu.make_async_copy(ea_h.at[pl.ds(0, 2 * CH)], eav[b],
                                  si[b]).wait()

        def issue_gathers(b):
            pltpu.async_copy(a_h.at[dstv[b]], arows[b], sg[b])
            pltpu.async_copy(b_h.at[srcv[b]], brows[b], sg[b])

        def wait_gathers(b):
            pltpu.make_async_copy(a_h.at[dstv[b]], arows[b], sg[b]).wait()
            pltpu.make_async_copy(b_h.at[srcv[b]], brows[b], sg[b]).wait()

        def wait_scatter(b):
            pltpu.make_async_copy(trows[b], acc.at[dstv[b]], ss[b]).wait()

        def compute(b):
            ar = arows[b]
            br = brows[b]
            tr = trows[b]
            ev = eav[b]

            def group(gi, gcarry):
                pv = ev[pl.ds(gi * 16, 16)]
                partner = _lane_gather(pv, pidx)
                d = (pv - partner) * 0.5
                nrm = d * _rsqrt_nr(d * d + EPS)
                row = gi * 8
                for j in range(8):
                    e = row + j
                    e0 = _lane_gather(nrm, jnp.full((16,), 2 * j, jnp.int32))
                    e1 = _lane_gather(nrm, jnp.full((16,), 2 * j + 1,
                                                    jnp.int32))
                    m = [ar[e, pl.ds(16 * t, 16)] + br[e, pl.ds(16 * t, 16)]
                         + e0 * we0[t] + e1 * we1[t] for t in range(4)]
                    s1 = _lane_sum(m[0] + m[1] + m[2] + m[3])
                    s2 = _lane_sum(m[0] * m[0] + m[1] * m[1]
                                   + m[2] * m[2] + m[3] * m[3])
                    mv = s1 * (1.0 / EMB)
                    var = s2 * (1.0 / EMB) - mv * mv
                    ry = _rsqrt_nr(var + EPS)
                    for t in range(4):
                        tt = (m[t] - mv) * ry * gv[t] + bv[t]
                        tr[e, pl.ds(16 * t, 16)] = jnp.maximum(tt, 0.0)
                return gcarry
            lax.fori_loop(0, CH // 8, group, 0)

        issue_idx(0, 0)
        wait_idx(0)
        issue_gathers(0)

        def quad(kq, carry):
            for b in range(NB):
                k = kq * NB + b
                nb = (b + 1) % NB
                if b == NB - 1:
                    wait_scatter(nb)
                else:
                    @pl.when(kq > 0)
                    def _():
                        wait_scatter(nb)
                if b < NB - 1:
                    issue_idx(k + 1, nb)
                else:
                    @pl.when(kq < NQ - 1)
                    def _():
                        issue_idx(k + 1, nb)
                wait_gathers(b)
                compute(b)
                pltpu.async_copy(trows[b], acc.at[dstv[b]], ss[b], add=True)
                if b < NB - 1:
                    wait_idx(nb)
                    issue_gathers(nb)
                else:
                    @pl.when(kq < NQ - 1)
                    def _():
                        wait_idx(nb)
                        issue_gathers(nb)
            return carry
        lax.fori_loop(0, NQ, quad, 0)
        for b in range(1, NB):
            wait_scatter(b)

        plsc.subcore_barrier()
        out_base = cid * NP + sid * ZROWS
        pltpu.sync_copy(acc.at[pl.ds(sid * ZROWS, ZROWS)],
                        out_h.at[pl.ds(out_base, ZROWS)])

    return sc_conv


def _build_sc_deg():
    mesh = plsc.VectorSubcoreMesh(core_axis_name="c", subcore_axis_name="s")

    @functools.partial(
        pl.kernel,
        mesh=mesh,
        out_type=[jax.ShapeDtypeStruct((2 * NP, 16), jnp.float32),
                  jax.ShapeDtypeStruct((2 * NP, 16), jnp.float32)],
        compiler_params=pltpu.CompilerParams(use_tc_tiling_on_sc=False),
        scratch_types=[
            pltpu.VMEM((CH,), jnp.int32),
            pltpu.VMEM((CH,), jnp.int32),
            pltpu.VMEM((CH, 16), jnp.float32),
            pltpu.VMEM_SHARED((NP, 16), jnp.float32),
            pltpu.VMEM_SHARED((NP, 16), jnp.float32),
        ],
    )
    def sc_deg(src_h, dst_h, degd_h, degs_h, srcv, dstv, buf, accd, accs):
        cid = lax.axis_index("c")
        sid = lax.axis_index("s")
        wid = cid * 16 + sid

        def _fill(val):
            def _row(r, carry):
                buf[r, pl.ds(0, 16)] = jnp.full((16,), val, jnp.float32)
                return carry
            lax.fori_loop(0, CH, _row, 0)

        _fill(0.0)
        base_r = sid * ZROWS
        for k in range(ZROWS // CH):
            pltpu.sync_copy(buf, accd.at[pl.ds(base_r + k * CH, CH)])
            pltpu.sync_copy(buf, accs.at[pl.ds(base_r + k * CH, CH)])
        _fill(1.0)
        plsc.subcore_barrier()

        ebase = wid * EPW

        def chunk(k, carry):
            off = ebase + k * CH
            pltpu.sync_copy(src_h.at[pl.ds(off, CH)], srcv)
            pltpu.sync_copy(dst_h.at[pl.ds(off, CH)], dstv)
            pltpu.sync_copy(buf, accd.at[dstv], add=True)
            pltpu.sync_copy(buf, accs.at[srcv], add=True)
            return carry
        lax.fori_loop(0, NCHUNK, chunk, 0)

        plsc.subcore_barrier()
        out_base = cid * NP + sid * ZROWS
        pltpu.sync_copy(accd.at[pl.ds(sid * ZROWS, ZROWS)],
                        degd_h.at[pl.ds(out_base, ZROWS)])
        pltpu.sync_copy(accs.at[pl.ds(sid * ZROWS, ZROWS)],
                        degs_h.at[pl.ds(out_base, ZROWS)])

    return sc_deg


_SC_CONV = _build_sc_conv()
_SC_DEG = _build_sc_deg()


def _ln64(x, g, b):
    m = jnp.mean(x, axis=-1, keepdims=True)
    v = jnp.mean((x - m) * (x - m), axis=-1, keepdims=True)
    return (x - m) * lax.rsqrt(v + EPS) * g + b


def _mm(a, b):
    return jnp.dot(a, b, preferred_element_type=jnp.float32)


def _ln2cols(ea):
    d = (ea[:, 0:1] - ea[:, 1:2]) * 0.5
    return d * lax.rsqrt(d * d + EPS)


def _embed_body(F, x_ref, lg_ref, lb_ref, w1_ref, b1_ref, w2_ref, b2_ref,
                o_ref):
    x = x_ref[...]
    mask = (lax.broadcasted_iota(jnp.int32, x.shape, 1) < F).astype(jnp.float32)
    m = jnp.sum(x * mask, axis=-1, keepdims=True) * (1.0 / F)
    v = jnp.sum(((x - m) * mask) ** 2, axis=-1, keepdims=True) * (1.0 / F)
    h = ((x - m) * lax.rsqrt(v + EPS) * lg_ref[...] + lb_ref[...]) * mask
    h = jnp.maximum(_mm(h, w1_ref[...]) + b1_ref[...], 0.0)
    o_ref[...] = jnp.maximum(_mm(h, w2_ref[...]) + b2_ref[...], 0.0)


def _embed(x_pad, ep, F, rows, grid):
    lg = jnp.pad(ep['ln_g'][None, :], ((0, 0), (0, 128 - F)))
    lb = jnp.pad(ep['ln_b'][None, :], ((0, 0), (0, 128 - F)))
    w1 = jnp.pad(ep['W1'].T, ((0, 128 - F), (0, 0)))
    return pl.pallas_call(
        functools.partial(_embed_body, F),
        grid=(grid,),
        in_specs=[
            pl.BlockSpec((rows, 128), lambda i: (i, 0)),
            pl.BlockSpec((1, 128), lambda i: (0, 0)),
            pl.BlockSpec((1, 128), lambda i: (0, 0)),
            pl.BlockSpec((128, EMB), lambda i: (0, 0)),
            pl.BlockSpec((1, EMB), lambda i: (0, 0)),
            pl.BlockSpec((EMB, EMB), lambda i: (0, 0)),
            pl.BlockSpec((1, EMB), lambda i: (0, 0)),
        ],
        out_specs=pl.BlockSpec((rows, EMB), lambda i: (i, 0)),
        out_shape=jax.ShapeDtypeStruct((rows * grid, EMB), jnp.float32),
    )(x_pad, lg, lb, w1, ep['b1'][None, :], ep['W2'].T, ep['b2'][None, :])


def _rowconv_body(x_ref, ea_ref, o_ref, wl_ref, bl_ref, we0_ref, we1_ref,
                  wr_ref, cg_ref, cb_ref, wf_ref, bf_ref, lg_ref, lb_ref,
                  w1a_ref, w1b_ref, b1_ref, w2_ref, b2_ref, out_ref):
    x = x_ref[...]
    n0 = _ln2cols(ea_ref[...])
    rvec = (_mm(o_ref[...], wr_ref[...]))[0:1]
    m = _mm(x, wl_ref[...]) + bl_ref[...] + n0 * (we0_ref[...] - we1_ref[...]) \
        + rvec
    m = jnp.maximum(_ln64(m, cg_ref[...], cb_ref[...]), 0.0)
    s = _mm(m, wf_ref[...]) + bf_ref[...]
    z = _ln64(s, lg_ref[...], lb_ref[...])
    h = jnp.maximum(_mm(x, w1a_ref[...]) + _mm(z, w1b_ref[...]) + b1_ref[...],
                    0.0)
    out_ref[...] = _mm(h, w2_ref[...]) + b2_ref[...]


def _reduceconv_body(x_ref, ea_ref, o_ref, wl_ref, bl_ref, we0_ref, we1_ref,
                     wr_ref, cg_ref, cb_ref, out_ref):
    i = pl.program_id(0)

    @pl.when(i == 0)
    def _():
        out_ref[...] = jnp.zeros_like(out_ref)

    x = x_ref[...]
    n0 = _ln2cols(ea_ref[...])
    lvec = (_mm(o_ref[...], wl_ref[...]) + bl_ref[...])[0:1]
    m = _mm(x, wr_ref[...]) + lvec + n0 * (we0_ref[...] - we1_ref[...])
    m = jnp.maximum(_ln64(m, cg_ref[...], cb_ref[...]), 0.0)
    rowid = i * BR + lax.broadcasted_iota(jnp.int32, (BR, 1), 0)
    m = m * (rowid < NN).astype(jnp.float32)
    out_ref[0:1, :] += jnp.sum(m, axis=0, keepdims=True)


def _objupdate_body(nreal, s_ref, o_ref, wf_ref, bf_ref, lg_ref, lb_ref,
                    w1a_ref, w1b_ref, b1_ref, w2_ref, b2_ref, out_ref):
    s = _mm(s_ref[...], wf_ref[...]) + nreal * bf_ref[...]
    z = _ln64(s, lg_ref[...], lb_ref[...])
    h = jnp.maximum(_mm(o_ref[...], w1a_ref[...]) + _mm(z, w1b_ref[...])
                    + b1_ref[...], 0.0)
    out_ref[...] = _mm(h, w2_ref[...]) + b2_ref[...]


def _ab_body(x_ref, y_ref, wl_ref, bl_ref, wr_ref, a_ref, b_ref):
    a_ref[...] = _mm(x_ref[...], wl_ref[...]) + bl_ref[...]
    b_ref[...] = _mm(y_ref[...], wr_ref[...])


def _post_body(a0_ref, a1_ref, d0_ref, d1_ref, x_ref, wf_ref, bf_ref,
               lg_ref, lb_ref, w1a_ref, w1b_ref, b1_ref, w2_ref, b2_ref,
               out_ref):
    agg = a0_ref[...] + a1_ref[...]
    deg = d0_ref[:, 0:1] + d1_ref[:, 0:1]
    s = _mm(agg, wf_ref[...]) + deg * bf_ref[...]
    z = _ln64(s, lg_ref[...], lb_ref[...])
    h = jnp.maximum(_mm(x_ref[...], w1a_ref[...]) + _mm(z, w1b_ref[...])
                    + b1_ref[...], 0.0)
    out_ref[...] = _mm(h, w2_ref[...]) + b2_ref[...]


def _head_body(x_ref, w1_ref, b1_ref, w2_ref, out_ref):
    h = jnp.maximum(_mm(x_ref[...], w1_ref[...]) + b1_ref[...], 0.0)
    out_ref[...] = jax.nn.sigmoid(_mm(h, w2_ref[...]))


def _wspec(r, c):
    return pl.BlockSpec((r, c), lambda i: (0, 0))


def _xspec(c):
    return pl.BlockSpec((BR, c), lambda i: (i, 0))


def _conv_we(cp, elg, elb):
    we = cp['We']
    we0 = (we[:, 0] * elg[0])[None, :]
    we1 = (we[:, 1] * elg[1])[None, :]
    ebias = we[:, 0] * elb[0] + we[:, 1] * elb[1]
    blv = (cp['bl'] + ebias)[None, :]
    return we0, we1, blv


def _rowconv(x, ea_pad, o8, cp, dp, lg, lb, we0, we1, blv):
    return pl.pallas_call(
        _rowconv_body,
        grid=(GRID,),
        in_specs=[
            _xspec(EMB), _xspec(128), _wspec(8, EMB),
            _wspec(EMB, EMB), _wspec(1, EMB), _wspec(1, EMB), _wspec(1, EMB),
            _wspec(EMB, EMB), _wspec(1, EMB), _wspec(1, EMB),
            _wspec(EMB, EMB), _wspec(1, EMB), _wspec(1, EMB), _wspec(1, EMB),
            _wspec(EMB, EMB), _wspec(EMB, EMB), _wspec(1, EMB),
            _wspec(EMB, EMB), _wspec(1, EMB),
        ],
        out_specs=_xspec(EMB),
        out_shape=jax.ShapeDtypeStruct((NP, EMB), jnp.float32),
    )(x, ea_pad, o8, cp['Wl'].T, blv, we0, we1, cp['Wr'].T,
      cp['g'][None, :], cp['bln'][None, :], cp['Wf'].T, cp['bf'][None, :],
      lg, lb, dp['W1'][:, :EMB].T, dp['W1'][:, EMB:].T, dp['b1'][None, :],
      dp['W2'].T, dp['b2'][None, :])


def _reduceconv(x, ea_pad, o8, cp, we0, we1, blv):
    return pl.pallas_call(
        _reduceconv_body,
        grid=(GRID,),
        in_specs=[
            _xspec(EMB), _xspec(128), _wspec(8, EMB),
            _wspec(EMB, EMB), _wspec(1, EMB), _wspec(1, EMB), _wspec(1, EMB),
            _wspec(EMB, EMB), _wspec(1, EMB), _wspec(1, EMB),
        ],
        out_specs=pl.BlockSpec((8, EMB), lambda i: (0, 0)),
        out_shape=jax.ShapeDtypeStruct((8, EMB), jnp.float32),
    )(x, ea_pad, o8, cp['Wl'].T, blv, we0, we1, cp['Wr'].T,
      cp['g'][None, :], cp['bln'][None, :])


def _objupdate(s8, o8, cp, dp, lg, lb, nreal):
    return pl.pallas_call(
        functools.partial(_objupdate_body, float(nreal)),
        grid=(1,),
        in_specs=[
            _wspec(8, EMB), _wspec(8, EMB),
            _wspec(EMB, EMB), _wspec(1, EMB), _wspec(1, EMB), _wspec(1, EMB),
            _wspec(EMB, EMB), _wspec(EMB, EMB), _wspec(1, EMB),
            _wspec(EMB, EMB), _wspec(1, EMB),
        ],
        out_specs=pl.BlockSpec((8, EMB), lambda i: (0, 0)),
        out_shape=jax.ShapeDtypeStruct((8, EMB), jnp.float32),
    )(s8, o8, cp['Wf'].T, cp['bf'][None, :], lg, lb,
      dp['W1'][:, :EMB].T, dp['W1'][:, EMB:].T, dp['b1'][None, :],
      dp['W2'].T, dp['b2'][None, :])


def _ab(x, y, cp, blv):
    return pl.pallas_call(
        _ab_body,
        grid=(GRID,),
        in_specs=[_xspec(EMB), _xspec(EMB),
                  _wspec(EMB, EMB), _wspec(1, EMB), _wspec(EMB, EMB)],
        out_specs=[_xspec(EMB), _xspec(EMB)],
        out_shape=[jax.ShapeDtypeStruct((NP, EMB), jnp.float32),
                   jax.ShapeDtypeStruct((NP, EMB), jnp.float32)],
    )(x, y, cp['Wl'].T, blv, cp['Wr'].T)


def _post(accp, degp, x, cp, dp, lg, lb):
    return pl.pallas_call(
        _post_body,
        grid=(GRID,),
        in_specs=[
            pl.BlockSpec((BR, EMB), lambda i: (i, 0)),
            pl.BlockSpec((BR, EMB), lambda i: (i + GRID, 0)),
            pl.BlockSpec((BR, 16), lambda i: (i, 0)),
            pl.BlockSpec((BR, 16), lambda i: (i + GRID, 0)),
            _xspec(EMB),
            _wspec(EMB, EMB), _wspec(1, EMB), _wspec(1, EMB), _wspec(1, EMB),
            _wspec(EMB, EMB), _wspec(EMB, EMB), _wspec(1, EMB),
            _wspec(EMB, EMB), _wspec(1, EMB),
        ],
        out_specs=_xspec(EMB),
        out_shape=jax.ShapeDtypeStruct((NP, EMB), jnp.float32),
    )(accp, accp, degp, degp, x, cp['Wf'].T, cp['bf'][None, :], lg, lb,
      dp['W1'][:, :EMB].T, dp['W1'][:, EMB:].T, dp['b1'][None, :],
      dp['W2'].T, dp['b2'][None, :])


def _head(x, w1t, b1, w2p):
    return pl.pallas_call(
        _head_body,
        grid=(GRID,),
        in_specs=[_xspec(EMB), _wspec(EMB, EMB), _wspec(1, EMB),
                  _wspec(EMB, 128)],
        out_specs=_xspec(128),
        out_shape=jax.ShapeDtypeStruct((NP, 128), jnp.float32),
    )(x, w1t, b1, w2p)


def kernel(x_u, x_c, x_o, ea_vc, ea_ov, ea_oc, ei_vc, ei_ov, ei_oc, params):
    p = params
    f32 = jnp.float32
    lg = p['ln_g'][None, :]
    lb = p['ln_b'][None, :]
    elg = p['edge_ln_g']
    elb = p['edge_ln_b']

    def padrc(a, rows, cols=128):
        return jnp.pad(a, ((0, rows - a.shape[0]), (0, cols - a.shape[1])))

    u = _embed(padrc(x_u, NP), p['ne0'], 14, BR, GRID)
    c = _embed(padrc(x_c, NP), p['ne1'], 6, BR, GRID)
    o = _embed(padrc(x_o, 8), p['ne2'], 2, 8, 1)

    ea_ov_pad = padrc(ea_ov, NP)
    ea_oc_pad = padrc(ea_oc, NP)

    vi = jnp.concatenate([ei_vc[0], jnp.full((EP - NE,), NN, jnp.int32)])
    ci = jnp.concatenate([ei_vc[1], jnp.full((EP - NE,), NN, jnp.int32)])
    eaf = jnp.concatenate([ea_vc, jnp.zeros((EP - NE, 2), f32)]).reshape(-1)

    degc, degv = _SC_DEG(vi, ci)

    for l in range(2):
        cp = p['conv%d_u_obj' % l]
        we0, we1, blv = _conv_we(cp, elg, elb)
        s8 = _reduceconv(u, ea_ov_pad, o, cp, we0, we1, blv)
        o = _objupdate(s8, o, cp, p['emb%d_obj' % l], lg, lb, NN)

        cp = p['conv%d_obj_con' % l]
        we0, we1, blv = _conv_we(cp, elg, elb)
        c = _rowconv(c, ea_oc_pad, o, cp, p['emb%d_con' % l], lg, lb,
                     we0, we1, blv)

        cp = p['conv%d_u_con' % l]
        we0, we1, blv = _conv_we(cp, elg, elb)
        a_t, b_t = _ab(c, u, cp, blv)
        wconst = jnp.concatenate([we0[0], we1[0], cp['g'], cp['bln']])
        accp = _SC_CONV(a_t, b_t, vi, ci, eaf, wconst)
        c = _post(accp, degc, c, cp, p['emb%d_con' % l], lg, lb)

        cp = p['conv%d_con_obj' % l]
        we0, we1, blv = _conv_we(cp, elg, elb)
        s8 = _reduceconv(c, ea_oc_pad, o, cp, we0, we1, blv)
        o = _objupdate(s8, o, cp, p['emb%d_obj' % l], lg, lb, NN)

        cp = p['conv%d_obj_u' % l]
        we0, we1, blv = _conv_we(cp, elg, elb)
        u = _rowconv(u, ea_ov_pad, o, cp, p['emb%d_u' % l], lg, lb,
                     we0, we1, blv)

        cp = p['conv%d_con_u' % l]
        we0, we1, blv = _conv_we(cp, elg, elb)
        a_t, b_t = _ab(u, c, cp, blv)
        wconst = jnp.concatenate([we0[0], we1[0], cp['g'], cp['bln']])
        accp = _SC_CONV(a_t, b_t, ci, vi, eaf, wconst)
        u = _post(accp, degv, u, cp, p['emb%d_u' % l], lg, lb)

    w2p = jnp.pad(p['out_W2'].T, ((0, 0), (0, 127)))
    res = _head(u, p['out_W1'].T, p['out_b1'][None, :], w2p)
    return res[:NN, :1]


# speedup vs baseline: 1.0756x; 1.0756x over previous
import functools

import jax
import jax.numpy as jnp
from jax import lax
from jax.experimental import pallas as pl
from jax.experimental.pallas import tpu as pltpu
from jax.experimental.pallas import tpu_sc as plsc

EMB = 64
NN = 10000
NP = 10240
NE = 320000
EP = 327680
NWORK = 32
EPW = EP // NWORK
CH = 128
NCHUNK = EPW // CH
ZROWS = NP // 16
BR = 1024
GRID = NP // BR
EPS = 1e-5


def _lane_gather(x, idx):
    dn = lax.GatherDimensionNumbers(
        offset_dims=(), collapsed_slice_dims=(0,), start_index_map=(0,))
    return lax.gather(x, idx[:, None], dn, (1,),
                      mode=lax.GatherScatterMode.PROMISE_IN_BOUNDS)


def _lane_sum(x):
    idx = lax.iota(jnp.int32, 16)
    for sh in (8, 4, 2, 1):
        x = x + _lane_gather(x, idx ^ sh)
    return x


def _rsqrt_nr(v):
    i = lax.bitcast_convert_type(v, jnp.int32)
    i = jnp.int32(0x5F3759DF) - lax.shift_right_logical(i, 1)
    y = lax.bitcast_convert_type(i, jnp.float32)
    for _ in range(3):
        y = y * (1.5 - 0.5 * v * y * y)
    return y


def _build_sc_conv():
    mesh = plsc.VectorSubcoreMesh(core_axis_name="c", subcore_axis_name="s")

    NB = 2
    NQ = NCHUNK // NB
    bufset = [pltpu.VMEM((CH,), jnp.int32), pltpu.VMEM((CH,), jnp.int32),
              pltpu.VMEM((2 * CH,), jnp.float32),
              pltpu.VMEM((CH, EMB), jnp.float32),
              pltpu.VMEM((CH, EMB), jnp.float32),
              pltpu.VMEM((CH, EMB), jnp.float32),
              pltpu.VMEM((CH,), jnp.int32),
              pltpu.SemaphoreType.DMA, pltpu.SemaphoreType.DMA,
              pltpu.SemaphoreType.DMA]

    @functools.partial(
        pl.kernel,
        mesh=mesh,
        out_type=jax.ShapeDtypeStruct((2 * NP, EMB), jnp.float32),
        compiler_params=pltpu.CompilerParams(use_tc_tiling_on_sc=False),
        scratch_types=(bufset * NB) + [
            pltpu.VMEM((4 * EMB,), jnp.float32),
            pltpu.VMEM_SHARED((NP, EMB), jnp.float32),
        ],
    )
    def sc_conv(a_h, b_h, src_h, dst_h, ea_h, w_h, out_h, *scr):
        wv, acc = scr[-2], scr[-1]
        srcv = tuple(scr[10 * b + 0] for b in range(NB))
        dstv = tuple(scr[10 * b + 1] for b in range(NB))
        eav = tuple(scr[10 * b + 2] for b in range(NB))
        arows = tuple(scr[10 * b + 3] for b in range(NB))
        brows = tuple(scr[10 * b + 4] for b in range(NB))
        trows = tuple(scr[10 * b + 5] for b in range(NB))
        sdix = tuple(scr[10 * b + 6] for b in range(NB))
        si = tuple(scr[10 * b + 7] for b in range(NB))
        sg = tuple(scr[10 * b + 8] for b in range(NB))
        ss = tuple(scr[10 * b + 9] for b in range(NB))
        trows0 = trows[0]
        cid = lax.axis_index("c")
        sid = lax.axis_index("s")
        wid = cid * 16 + sid

        def _zrow(r, carry):
            zero = jnp.zeros((16,), jnp.float32)
            for t in range(4):
                trows0[r, pl.ds(16 * t, 16)] = zero
            return carry
        lax.fori_loop(0, CH, _zrow, 0)
        base_r = sid * ZROWS
        for k in range(ZROWS // CH):
            pltpu.sync_copy(trows0, acc.at[pl.ds(base_r + k * CH, CH)])
        plsc.subcore_barrier()

        pltpu.sync_copy(w_h, wv)
        we0 = [wv[pl.ds(16 * t, 16)] for t in range(4)]
        we1 = [wv[pl.ds(EMB + 16 * t, 16)] for t in range(4)]
        gv = [wv[pl.ds(2 * EMB + 16 * t, 16)] for t in range(4)]
        bv = [wv[pl.ds(3 * EMB + 16 * t, 16)] for t in range(4)]
        pidx = lax.iota(jnp.int32, 16) ^ 1

        ebase = wid * EPW

        def issue_idx(k, b):
            off = ebase + k * CH
            pltpu.async_copy(src_h.at[pl.ds(off, CH)], srcv[b], si[b])
            pltpu.async_copy(dst_h.at[pl.ds(off, CH)], dstv[b], si[b])
            pltpu.async_copy(ea_h.at[pl.ds(2 * off, 2 * CH)], eav[b], si[b])

        def wait_idx(b):
            pltpu.make_async_copy(src_h.at[pl.ds(0, CH)], srcv[b],
                                  si[b]).wait()
            pltpu.make_async_copy(dst_h.at[pl.ds(0, CH)], dstv[b],
                                  si[b]).wait()
            pltpu.make_async_copy(ea_h.at[pl.ds(0, 2 * CH)], eav[b],
                                  si[b]).wait()

        def issue_gathers(b):
            pltpu.async_copy(a_h.at[dstv[b]], arows[b], sg[b])
            pltpu.async_copy(b_h.at[srcv[b]], brows[b], sg[b])

        def wait_gathers(b):
            pltpu.make_async_copy(a_h.at[dstv[b]], arows[b], sg[b]).wait()
            pltpu.make_async_copy(b_h.at[srcv[b]], brows[b], sg[b]).wait()

        def wait_scatter(b):
            pltpu.make_async_copy(trows[b], acc.at[sdix[b]], ss[b]).wait()

        def compute(b):
            ar = arows[b]
            br = brows[b]
            tr = trows[b]
            ev = eav[b]

            def group(gi, gcarry):
                pv = ev[pl.ds(gi * 16, 16)]
                partner = _lane_gather(pv, pidx)
                d = (pv - partner) * 0.5
                nrm = d * _rsqrt_nr(d * d + EPS)
                row = gi * 8
                for j in range(8):
                    e = row + j
                    e0 = _lane_gather(nrm, jnp.full((16,), 2 * j, jnp.int32))
                    e1 = _lane_gather(nrm, jnp.full((16,), 2 * j + 1,
                                                    jnp.int32))
                    m = [ar[e, pl.ds(16 * t, 16)] + br[e, pl.ds(16 * t, 16)]
                         + e0 * we0[t] + e1 * we1[t] for t in range(4)]
                    s1 = _lane_sum(m[0] + m[1] + m[2] + m[3])
                    s2 = _lane_sum(m[0] * m[0] + m[1] * m[1]
                                   + m[2] * m[2] + m[3] * m[3])
                    mv = s1 * (1.0 / EMB)
                    var = s2 * (1.0 / EMB) - mv * mv
                    ry = _rsqrt_nr(var + EPS)
                    for t in range(4):
                        tt = (m[t] - mv) * ry * gv[t] + bv[t]
                        tr[e, pl.ds(16 * t, 16)] = jnp.maximum(tt, 0.0)
                return gcarry
            lax.fori_loop(0, CH // 8, group, 0)

        issue_idx(0, 0)
        wait_idx(0)
        issue_gathers(0)

        def quad(kq, carry):
            for b in range(NB):
                k = kq * NB + b
                nb = (b + 1) % NB
                @pl.when(kq > 0)
                def _():
                    wait_scatter(b)
                if b < NB - 1:
                    issue_idx(k + 1, nb)
                else:
                    @pl.when(kq < NQ - 1)
                    def _():
                        issue_idx(k + 1, nb)
                wait_gathers(b)
                compute(b)
                for t in range(CH // 16):
                    sdix[b][pl.ds(16 * t, 16)] = dstv[b][pl.ds(16 * t, 16)]
                pltpu.async_copy(trows[b], acc.at[sdix[b]], ss[b], add=True)
                if b < NB - 1:
                    wait_idx(nb)
                    issue_gathers(nb)
                else:
                    @pl.when(kq < NQ - 1)
                    def _():
                        wait_idx(nb)
                        issue_gathers(nb)
            return carry
        lax.fori_loop(0, NQ, quad, 0)
        for b in range(NB):
            wait_scatter(b)

        plsc.subcore_barrier()
        out_base = cid * NP + sid * ZROWS
        pltpu.sync_copy(acc.at[pl.ds(sid * ZROWS, ZROWS)],
                        out_h.at[pl.ds(out_base, ZROWS)])

    return sc_conv


def _build_sc_deg():
    mesh = plsc.VectorSubcoreMesh(core_axis_name="c", subcore_axis_name="s")

    @functools.partial(
        pl.kernel,
        mesh=mesh,
        out_type=[jax.ShapeDtypeStruct((2 * NP, 16), jnp.float32),
                  jax.ShapeDtypeStruct((2 * NP, 16), jnp.float32)],
        compiler_params=pltpu.CompilerParams(use_tc_tiling_on_sc=False),
        scratch_types=[
            pltpu.VMEM((CH,), jnp.int32),
            pltpu.VMEM((CH,), jnp.int32),
            pltpu.VMEM((CH, 16), jnp.float32),
            pltpu.VMEM_SHARED((NP, 16), jnp.float32),
            pltpu.VMEM_SHARED((NP, 16), jnp.float32),
        ],
    )
    def sc_deg(src_h, dst_h, degd_h, degs_h, srcv, dstv, buf, accd, accs):
        cid = lax.axis_index("c")
        sid = lax.axis_index("s")
        wid = cid * 16 + sid

        def _fill(val):
            def _row(r, carry):
                buf[r, pl.ds(0, 16)] = jnp.full((16,), val, jnp.float32)
                return carry
            lax.fori_loop(0, CH, _row, 0)

        _fill(0.0)
        base_r = sid * ZROWS
        for k in range(ZROWS // CH):
            pltpu.sync_copy(buf, accd.at[pl.ds(base_r + k * CH, CH)])
            pltpu.sync_copy(buf, accs.at[pl.ds(base_r + k * CH, CH)])
        _fill(1.0)
        plsc.subcore_barrier()

        ebase = wid * EPW

        def chunk(k, carry):
            off = ebase + k * CH
            pltpu.sync_copy(src_h.at[pl.ds(off, CH)], srcv)
            pltpu.sync_copy(dst_h.at[pl.ds(off, CH)], dstv)
            pltpu.sync_copy(buf, accd.at[dstv], add=True)
            pltpu.sync_copy(buf, accs.at[srcv], add=True)
            return carry
        lax.fori_loop(0, NCHUNK, chunk, 0)

        plsc.subcore_barrier()
        out_base = cid * NP + sid * ZROWS
        pltpu.sync_copy(accd.at[pl.ds(sid * ZROWS, ZROWS)],
                        degd_h.at[pl.ds(out_base, ZROWS)])
        pltpu.sync_copy(accs.at[pl.ds(sid * ZROWS, ZROWS)],
                        degs_h.at[pl.ds(out_base, ZROWS)])

    return sc_deg


_SC_CONV = _build_sc_conv()
_SC_DEG = _build_sc_deg()


def _ln64(x, g, b):
    m = jnp.mean(x, axis=-1, keepdims=True)
    v = jnp.mean((x - m) * (x - m), axis=-1, keepdims=True)
    return (x - m) * lax.rsqrt(v + EPS) * g + b


def _mm(a, b):
    return jnp.dot(a, b, preferred_element_type=jnp.float32)


def _ln2cols(ea):
    d = (ea[:, 0:1] - ea[:, 1:2]) * 0.5
    return d * lax.rsqrt(d * d + EPS)


def _embed_body(F, x_ref, lg_ref, lb_ref, w1_ref, b1_ref, w2_ref, b2_ref,
                o_ref):
    x = x_ref[...]
    mask = (lax.broadcasted_iota(jnp.int32, x.shape, 1) < F).astype(jnp.float32)
    m = jnp.sum(x * mask, axis=-1, keepdims=True) * (1.0 / F)
    v = jnp.sum(((x - m) * mask) ** 2, axis=-1, keepdims=True) * (1.0 / F)
    h = ((x - m) * lax.rsqrt(v + EPS) * lg_ref[...] + lb_ref[...]) * mask
    h = jnp.maximum(_mm(h, w1_ref[...]) + b1_ref[...], 0.0)
    o_ref[...] = jnp.maximum(_mm(h, w2_ref[...]) + b2_ref[...], 0.0)


def _embed(x_pad, ep, F, rows, grid):
    lg = jnp.pad(ep['ln_g'][None, :], ((0, 0), (0, 128 - F)))
    lb = jnp.pad(ep['ln_b'][None, :], ((0, 0), (0, 128 - F)))
    w1 = jnp.pad(ep['W1'].T, ((0, 128 - F), (0, 0)))
    return pl.pallas_call(
        functools.partial(_embed_body, F),
        grid=(grid,),
        in_specs=[
            pl.BlockSpec((rows, 128), lambda i: (i, 0)),
            pl.BlockSpec((1, 128), lambda i: (0, 0)),
            pl.BlockSpec((1, 128), lambda i: (0, 0)),
            pl.BlockSpec((128, EMB), lambda i: (0, 0)),
            pl.BlockSpec((1, EMB), lambda i: (0, 0)),
            pl.BlockSpec((EMB, EMB), lambda i: (0, 0)),
            pl.BlockSpec((1, EMB), lambda i: (0, 0)),
        ],
        out_specs=pl.BlockSpec((rows, EMB), lambda i: (i, 0)),
        out_shape=jax.ShapeDtypeStruct((rows * grid, EMB), jnp.float32),
    )(x_pad, lg, lb, w1, ep['b1'][None, :], ep['W2'].T, ep['b2'][None, :])


def _rowconv_body(x_ref, ea_ref, o_ref, wl_ref, bl_ref, we0_ref, we1_ref,
                  wr_ref, cg_ref, cb_ref, wf_ref, bf_ref, lg_ref, lb_ref,
                  w1a_ref, w1b_ref, b1_ref, w2_ref, b2_ref, out_ref):
    x = x_ref[...]
    n0 = _ln2cols(ea_ref[...])
    rvec = (_mm(o_ref[...], wr_ref[...]))[0:1]
    m = _mm(x, wl_ref[...]) + bl_ref[...] + n0 * (we0_ref[...] - we1_ref[...]) \
        + rvec
    m = jnp.maximum(_ln64(m, cg_ref[...], cb_ref[...]), 0.0)
    s = _mm(m, wf_ref[...]) + bf_ref[...]
    z = _ln64(s, lg_ref[...], lb_ref[...])
    h = jnp.maximum(_mm(x, w1a_ref[...]) + _mm(z, w1b_ref[...]) + b1_ref[...],
                    0.0)
    out_ref[...] = _mm(h, w2_ref[...]) + b2_ref[...]


def _reduceconv_body(x_ref, ea_ref, o_ref, wl_ref, bl_ref, we0_ref, we1_ref,
                     wr_ref, cg_ref, cb_ref, out_ref):
    i = pl.program_id(0)

    @pl.when(i == 0)
    def _():
        out_ref[...] = jnp.zeros_like(out_ref)

    x = x_ref[...]
    n0 = _ln2cols(ea_ref[...])
    lvec = (_mm(o_ref[...], wl_ref[...]) + bl_ref[...])[0:1]
    m = _mm(x, wr_ref[...]) + lvec + n0 * (we0_ref[...] - we1_ref[...])
    m = jnp.maximum(_ln64(m, cg_ref[...], cb_ref[...]), 0.0)
    rowid = i * BR + lax.broadcasted_iota(jnp.int32, (BR, 1), 0)
    m = m * (rowid < NN).astype(jnp.float32)
    out_ref[0:1, :] += jnp.sum(m, axis=0, keepdims=True)


def _objupdate_body(nreal, s_ref, o_ref, wf_ref, bf_ref, lg_ref, lb_ref,
                    w1a_ref, w1b_ref, b1_ref, w2_ref, b2_ref, out_ref):
    s = _mm(s_ref[...], wf_ref[...]) + nreal * bf_ref[...]
    z = _ln64(s, lg_ref[...], lb_ref[...])
    h = jnp.maximum(_mm(o_ref[...], w1a_ref[...]) + _mm(z, w1b_ref[...])
                    + b1_ref[...], 0.0)
    out_ref[...] = _mm(h, w2_ref[...]) + b2_ref[...]


def _ab_body(x_ref, y_ref, wl_ref, bl_ref, wr_ref, a_ref, b_ref):
    a_ref[...] = _mm(x_ref[...], wl_ref[...]) + bl_ref[...]
    b_ref[...] = _mm(y_ref[...], wr_ref[...])


def _post_body(a0_ref, a1_ref, d0_ref, d1_ref, x_ref, wf_ref, bf_ref,
               lg_ref, lb_ref, w1a_ref, w1b_ref, b1_ref, w2_ref, b2_ref,
               out_ref):
    agg = a0_ref[...] + a1_ref[...]
    deg = d0_ref[:, 0:1] + d1_ref[:, 0:1]
    s = _mm(agg, wf_ref[...]) + deg * bf_ref[...]
    z = _ln64(s, lg_ref[...], lb_ref[...])
    h = jnp.maximum(_mm(x_ref[...], w1a_ref[...]) + _mm(z, w1b_ref[...])
                    + b1_ref[...], 0.0)
    out_ref[...] = _mm(h, w2_ref[...]) + b2_ref[...]


def _head_body(x_ref, w1_ref, b1_ref, w2_ref, out_ref):
    h = jnp.maximum(_mm(x_ref[...], w1_ref[...]) + b1_ref[...], 0.0)
    out_ref[...] = jax.nn.sigmoid(_mm(h, w2_ref[...]))


def _wspec(r, c):
    return pl.BlockSpec((r, c), lambda i: (0, 0))


def _xspec(c):
    return pl.BlockSpec((BR, c), lambda i: (i, 0))


def _conv_we(cp, elg, elb):
    we = cp['We']
    we0 = (we[:, 0] * elg[0])[None, :]
    we1 = (we[:, 1] * elg[1])[None, :]
    ebias = we[:, 0] * elb[0] + we[:, 1] * elb[1]
    blv = (cp['bl'] + ebias)[None, :]
    return we0, we1, blv


def _rowconv(x, ea_pad, o8, cp, dp, lg, lb, we0, we1, blv):
    return pl.pallas_call(
        _rowconv_body,
        grid=(GRID,),
        in_specs=[
            _xspec(EMB), _xspec(128), _wspec(8, EMB),
            _wspec(EMB, EMB), _wspec(1, EMB), _wspec(1, EMB), _wspec(1, EMB),
            _wspec(EMB, EMB), _wspec(1, EMB), _wspec(1, EMB),
            _wspec(EMB, EMB), _wspec(1, EMB), _wspec(1, EMB), _wspec(1, EMB),
            _wspec(EMB, EMB), _wspec(EMB, EMB), _wspec(1, EMB),
            _wspec(EMB, EMB), _wspec(1, EMB),
        ],
        out_specs=_xspec(EMB),
        out_shape=jax.ShapeDtypeStruct((NP, EMB), jnp.float32),
    )(x, ea_pad, o8, cp['Wl'].T, blv, we0, we1, cp['Wr'].T,
      cp['g'][None, :], cp['bln'][None, :], cp['Wf'].T, cp['bf'][None, :],
      lg, lb, dp['W1'][:, :EMB].T, dp['W1'][:, EMB:].T, dp['b1'][None, :],
      dp['W2'].T, dp['b2'][None, :])


def _reduceconv(x, ea_pad, o8, cp, we0, we1, blv):
    return pl.pallas_call(
        _reduceconv_body,
        grid=(GRID,),
        in_specs=[
            _xspec(EMB), _xspec(128), _wspec(8, EMB),
            _wspec(EMB, EMB), _wspec(1, EMB), _wspec(1, EMB), _wspec(1, EMB),
            _wspec(EMB, EMB), _wspec(1, EMB), _wspec(1, EMB),
        ],
        out_specs=pl.BlockSpec((8, EMB), lambda i: (0, 0)),
        out_shape=jax.ShapeDtypeStruct((8, EMB), jnp.float32),
    )(x, ea_pad, o8, cp['Wl'].T, blv, we0, we1, cp['Wr'].T,
      cp['g'][None, :], cp['bln'][None, :])


def _objupdate(s8, o8, cp, dp, lg, lb, nreal):
    return pl.pallas_call(
        functools.partial(_objupdate_body, float(nreal)),
        grid=(1,),
        in_specs=[
            _wspec(8, EMB), _wspec(8, EMB),
            _wspec(EMB, EMB), _wspec(1, EMB), _wspec(1, EMB), _wspec(1, EMB),
            _wspec(EMB, EMB), _wspec(EMB, EMB), _wspec(1, EMB),
            _wspec(EMB, EMB), _wspec(1, EMB),
        ],
        out_specs=pl.BlockSpec((8, EMB), lambda i: (0, 0)),
        out_shape=jax.ShapeDtypeStruct((8, EMB), jnp.float32),
    )(s8, o8, cp['Wf'].T, cp['bf'][None, :], lg, lb,
      dp['W1'][:, :EMB].T, dp['W1'][:, EMB:].T, dp['b1'][None, :],
      dp['W2'].T, dp['b2'][None, :])


def _ab(x, y, cp, blv):
    return pl.pallas_call(
        _ab_body,
        grid=(GRID,),
        in_specs=[_xspec(EMB), _xspec(EMB),
                  _wspec(EMB, EMB), _wspec(1, EMB), _wspec(EMB, EMB)],
        out_specs=[_xspec(EMB), _xspec(EMB)],
        out_shape=[jax.ShapeDtypeStruct((NP, EMB), jnp.float32),
                   jax.ShapeDtypeStruct((NP, EMB), jnp.float32)],
    )(x, y, cp['Wl'].T, blv, cp['Wr'].T)


def _post(accp, degp, x, cp, dp, lg, lb):
    return pl.pallas_call(
        _post_body,
        grid=(GRID,),
        in_specs=[
            pl.BlockSpec((BR, EMB), lambda i: (i, 0)),
            pl.BlockSpec((BR, EMB), lambda i: (i + GRID, 0)),
            pl.BlockSpec((BR, 16), lambda i: (i, 0)),
            pl.BlockSpec((BR, 16), lambda i: (i + GRID, 0)),
            _xspec(EMB),
            _wspec(EMB, EMB), _wspec(1, EMB), _wspec(1, EMB), _wspec(1, EMB),
            _wspec(EMB, EMB), _wspec(EMB, EMB), _wspec(1, EMB),
            _wspec(EMB, EMB), _wspec(1, EMB),
        ],
        out_specs=_xspec(EMB),
        out_shape=jax.ShapeDtypeStruct((NP, EMB), jnp.float32),
    )(accp, accp, degp, degp, x, cp['Wf'].T, cp['bf'][None, :], lg, lb,
      dp['W1'][:, :EMB].T, dp['W1'][:, EMB:].T, dp['b1'][None, :],
      dp['W2'].T, dp['b2'][None, :])


def _head(x, w1t, b1, w2p):
    return pl.pallas_call(
        _head_body,
        grid=(GRID,),
        in_specs=[_xspec(EMB), _wspec(EMB, EMB), _wspec(1, EMB),
                  _wspec(EMB, 128)],
        out_specs=_xspec(128),
        out_shape=jax.ShapeDtypeStruct((NP, 128), jnp.float32),
    )(x, w1t, b1, w2p)


def kernel(x_u, x_c, x_o, ea_vc, ea_ov, ea_oc, ei_vc, ei_ov, ei_oc, params):
    p = params
    f32 = jnp.float32
    lg = p['ln_g'][None, :]
    lb = p['ln_b'][None, :]
    elg = p['edge_ln_g']
    elb = p['edge_ln_b']

    def padrc(a, rows, cols=128):
        return jnp.pad(a, ((0, rows - a.shape[0]), (0, cols - a.shape[1])))

    u = _embed(padrc(x_u, NP), p['ne0'], 14, BR, GRID)
    c = _embed(padrc(x_c, NP), p['ne1'], 6, BR, GRID)
    o = _embed(padrc(x_o, 8), p['ne2'], 2, 8, 1)

    ea_ov_pad = padrc(ea_ov, NP)
    ea_oc_pad = padrc(ea_oc, NP)

    vi = jnp.concatenate([ei_vc[0], jnp.full((EP - NE,), NN, jnp.int32)])
    ci = jnp.concatenate([ei_vc[1], jnp.full((EP - NE,), NN, jnp.int32)])
    eaf = jnp.concatenate([ea_vc, jnp.zeros((EP - NE, 2), f32)]).reshape(-1)

    degc, degv = _SC_DEG(vi, ci)

    for l in range(2):
        cp = p['conv%d_u_obj' % l]
        we0, we1, blv = _conv_we(cp, elg, elb)
        s8 = _reduceconv(u, ea_ov_pad, o, cp, we0, we1, blv)
        o = _objupdate(s8, o, cp, p['emb%d_obj' % l], lg, lb, NN)

        cp = p['conv%d_obj_con' % l]
        we0, we1, blv = _conv_we(cp, elg, elb)
        c = _rowconv(c, ea_oc_pad, o, cp, p['emb%d_con' % l], lg, lb,
                     we0, we1, blv)

        cp = p['conv%d_u_con' % l]
        we0, we1, blv = _conv_we(cp, elg, elb)
        a_t, b_t = _ab(c, u, cp, blv)
        wconst = jnp.concatenate([we0[0], we1[0], cp['g'], cp['bln']])
        accp = _SC_CONV(a_t, b_t, vi, ci, eaf, wconst)
        c = _post(accp, degc, c, cp, p['emb%d_con' % l], lg, lb)

        cp = p['conv%d_con_obj' % l]
        we0, we1, blv = _conv_we(cp, elg, elb)
        s8 = _reduceconv(c, ea_oc_pad, o, cp, we0, we1, blv)
        o = _objupdate(s8, o, cp, p['emb%d_obj' % l], lg, lb, NN)

        cp = p['conv%d_obj_u' % l]
        we0, we1, blv = _conv_we(cp, elg, elb)
        u = _rowconv(u, ea_ov_pad, o, cp, p['emb%d_u' % l], lg, lb,
                     we0, we1, blv)

        cp = p['conv%d_con_u' % l]
        we0, we1, blv = _conv_we(cp, elg, elb)
        a_t, b_t = _ab(u, c, cp, blv)
        wconst = jnp.concatenate([we0[0], we1[0], cp['g'], cp['bln']])
        accp = _SC_CONV(a_t, b_t, ci, vi, eaf, wconst)
        u = _post(accp, degv, u, cp, p['emb%d_u' % l], lg, lb)

    w2p = jnp.pad(p['out_W2'].T, ((0, 0), (0, 127)))
    res = _head(u, p['out_W1'].T, p['out_b1'][None, :], w2p)
    return res[:NN, :1]


# speedup vs baseline: 1.1138x; 1.0354x over previous
import functools

import jax
import jax.numpy as jnp
from jax import lax
from jax.experimental import pallas as pl
from jax.experimental.pallas import tpu as pltpu
from jax.experimental.pallas import tpu_sc as plsc

EMB = 64
NN = 10000
NP = 10240
NE = 320000
EP = 327680
NWORK = 32
EPW = EP // NWORK
CH = 128
NCHUNK = EPW // CH
ZROWS = NP // 16
BR = 1024
GRID = NP // BR
EPS = 1e-5


def _lane_gather(x, idx):
    dn = lax.GatherDimensionNumbers(
        offset_dims=(), collapsed_slice_dims=(0,), start_index_map=(0,))
    return lax.gather(x, idx[:, None], dn, (1,),
                      mode=lax.GatherScatterMode.PROMISE_IN_BOUNDS)


def _lane_sum(x):
    idx = lax.iota(jnp.int32, 16)
    for sh in (8, 4, 2, 1):
        x = x + _lane_gather(x, idx ^ sh)
    return x


def _rsqrt_nr(v):
    i = lax.bitcast_convert_type(v, jnp.int32)
    i = jnp.int32(0x5F3759DF) - lax.shift_right_logical(i, 1)
    y = lax.bitcast_convert_type(i, jnp.float32)
    for _ in range(3):
        y = y * (1.5 - 0.5 * v * y * y)
    return y


def _build_sc_conv():
    mesh = plsc.VectorSubcoreMesh(core_axis_name="c", subcore_axis_name="s")

    NB = 2
    NQ = NCHUNK // NB
    bufset = [pltpu.VMEM((CH,), jnp.int32), pltpu.VMEM((CH,), jnp.int32),
              pltpu.VMEM((2 * CH,), jnp.float32),
              pltpu.VMEM((CH, EMB), jnp.float32),
              pltpu.VMEM((CH, EMB), jnp.float32),
              pltpu.VMEM((CH, EMB), jnp.float32),
              pltpu.VMEM((CH,), jnp.int32),
              pltpu.SemaphoreType.DMA, pltpu.SemaphoreType.DMA,
              pltpu.SemaphoreType.DMA]

    @functools.partial(
        pl.kernel,
        mesh=mesh,
        out_type=jax.ShapeDtypeStruct((2 * NP, EMB), jnp.float32),
        compiler_params=pltpu.CompilerParams(use_tc_tiling_on_sc=False),
        scratch_types=(bufset * NB) + [
            pltpu.VMEM((4 * EMB,), jnp.float32),
            pltpu.VMEM_SHARED((NP, EMB), jnp.float32),
        ],
    )
    def sc_conv(a_h, b_h, src_h, dst_h, ea_h, w_h, out_h, *scr):
        wv, acc = scr[-2], scr[-1]
        srcv = tuple(scr[10 * b + 0] for b in range(NB))
        dstv = tuple(scr[10 * b + 1] for b in range(NB))
        eav = tuple(scr[10 * b + 2] for b in range(NB))
        arows = tuple(scr[10 * b + 3] for b in range(NB))
        brows = tuple(scr[10 * b + 4] for b in range(NB))
        trows = tuple(scr[10 * b + 5] for b in range(NB))
        sdix = tuple(scr[10 * b + 6] for b in range(NB))
        si = tuple(scr[10 * b + 7] for b in range(NB))
        sg = tuple(scr[10 * b + 8] for b in range(NB))
        ss = tuple(scr[10 * b + 9] for b in range(NB))
        trows0 = trows[0]
        cid = lax.axis_index("c")
        sid = lax.axis_index("s")
        wid = cid * 16 + sid

        def _zrow(r, carry):
            zero = jnp.zeros((16,), jnp.float32)
            for t in range(4):
                trows0[r, pl.ds(16 * t, 16)] = zero
            return carry
        lax.fori_loop(0, CH, _zrow, 0)
        base_r = sid * ZROWS
        for k in range(ZROWS // CH):
            pltpu.sync_copy(trows0, acc.at[pl.ds(base_r + k * CH, CH)])
        plsc.subcore_barrier()

        pltpu.sync_copy(w_h, wv)
        we0 = [wv[pl.ds(16 * t, 16)] for t in range(4)]
        we1 = [wv[pl.ds(EMB + 16 * t, 16)] for t in range(4)]
        gv = [wv[pl.ds(2 * EMB + 16 * t, 16)] for t in range(4)]
        bv = [wv[pl.ds(3 * EMB + 16 * t, 16)] for t in range(4)]
        pidx = lax.iota(jnp.int32, 16) ^ 1

        ebase = wid * EPW

        def issue_idx(k, b):
            off = ebase + k * CH
            pltpu.async_copy(src_h.at[pl.ds(off, CH)], srcv[b], si[b])
            pltpu.async_copy(dst_h.at[pl.ds(off, CH)], dstv[b], si[b])
            pltpu.async_copy(ea_h.at[pl.ds(2 * off, 2 * CH)], eav[b], si[b])

        def wait_idx(b):
            pltpu.make_async_copy(src_h.at[pl.ds(0, CH)], srcv[b],
                                  si[b]).wait()
            pltpu.make_async_copy(dst_h.at[pl.ds(0, CH)], dstv[b],
                                  si[b]).wait()
            pltpu.make_async_copy(ea_h.at[pl.ds(0, 2 * CH)], eav[b],
                                  si[b]).wait()

        def issue_gathers(b):
            pltpu.async_copy(a_h.at[dstv[b]], arows[b], sg[b])
            pltpu.async_copy(b_h.at[srcv[b]], brows[b], sg[b])

        def wait_gathers(b):
            pltpu.make_async_copy(a_h.at[dstv[b]], arows[b], sg[b]).wait()
            pltpu.make_async_copy(b_h.at[srcv[b]], brows[b], sg[b]).wait()

        def wait_scatter(b):
            pltpu.make_async_copy(trows[b], acc.at[sdix[b]], ss[b]).wait()

        def compute(b):
            ar = arows[b]
            br = brows[b]
            tr = trows[b]
            ev = eav[b]

            def group(gi, gcarry):
                pv = ev[pl.ds(gi * 16, 16)]
                partner = _lane_gather(pv, pidx)
                d = (pv - partner) * 0.5
                nrm = d * _rsqrt_nr(d * d + EPS)
                row = gi * 8
                for j in range(8):
                    e = row + j
                    e0 = _lane_gather(nrm, jnp.full((16,), 2 * j, jnp.int32))
                    e1 = _lane_gather(nrm, jnp.full((16,), 2 * j + 1,
                                                    jnp.int32))
                    m = [ar[e, pl.ds(16 * t, 16)] + br[e, pl.ds(16 * t, 16)]
                         + e0 * we0[t] + e1 * we1[t] for t in range(4)]
                    s1 = _lane_sum(m[0] + m[1] + m[2] + m[3])
                    s2 = _lane_sum(m[0] * m[0] + m[1] * m[1]
                                   + m[2] * m[2] + m[3] * m[3])
                    mv = s1 * (1.0 / EMB)
                    var = s2 * (1.0 / EMB) - mv * mv
                    ry = _rsqrt_nr(var + EPS)
                    for t in range(4):
                        tt = (m[t] - mv) * ry * gv[t] + bv[t]
                        tr[e, pl.ds(16 * t, 16)] = jnp.maximum(tt, 0.0)
                return gcarry
            lax.fori_loop(0, CH // 8, group, 0)

        issue_idx(0, 0)
        wait_idx(0)
        issue_gathers(0)

        def quad(kq, carry):
            for b in range(NB):
                k = kq * NB + b
                nb = (b + 1) % NB
                @pl.when(kq > 0)
                def _():
                    wait_scatter(b)
                if b < NB - 1:
                    issue_idx(k + 1, nb)
                else:
                    @pl.when(kq < NQ - 1)
                    def _():
                        issue_idx(k + 1, nb)
                wait_gathers(b)
                compute(b)
                for t in range(CH // 16):
                    sdix[b][pl.ds(16 * t, 16)] = dstv[b][pl.ds(16 * t, 16)]
                pltpu.async_copy(trows[b], acc.at[sdix[b]], ss[b], add=True)
                if b < NB - 1:
                    wait_idx(nb)
                    issue_gathers(nb)
                else:
                    @pl.when(kq < NQ - 1)
                    def _():
                        wait_idx(nb)
                        issue_gathers(nb)
            return carry
        lax.fori_loop(0, NQ, quad, 0)
        for b in range(NB):
            wait_scatter(b)

        plsc.subcore_barrier()
        out_base = cid * NP + sid * ZROWS
        pltpu.sync_copy(acc.at[pl.ds(sid * ZROWS, ZROWS)],
                        out_h.at[pl.ds(out_base, ZROWS)])

    return sc_conv


def _build_sc_deg():
    mesh = plsc.VectorSubcoreMesh(core_axis_name="c", subcore_axis_name="s")

    @functools.partial(
        pl.kernel,
        mesh=mesh,
        out_type=[jax.ShapeDtypeStruct((2 * NP, 16), jnp.float32),
                  jax.ShapeDtypeStruct((2 * NP, 16), jnp.float32)],
        compiler_params=pltpu.CompilerParams(use_tc_tiling_on_sc=False),
        scratch_types=[
            pltpu.VMEM((CH,), jnp.int32),
            pltpu.VMEM((CH,), jnp.int32),
            pltpu.VMEM((CH, 16), jnp.float32),
            pltpu.VMEM_SHARED((NP, 16), jnp.float32),
            pltpu.VMEM_SHARED((NP, 16), jnp.float32),
        ],
    )
    def sc_deg(src_h, dst_h, degd_h, degs_h, srcv, dstv, buf, accd, accs):
        cid = lax.axis_index("c")
        sid = lax.axis_index("s")
        wid = cid * 16 + sid

        def _fill(val):
            def _row(r, carry):
                buf[r, pl.ds(0, 16)] = jnp.full((16,), val, jnp.float32)
                return carry
            lax.fori_loop(0, CH, _row, 0)

        _fill(0.0)
        base_r = sid * ZROWS
        for k in range(ZROWS // CH):
            pltpu.sync_copy(buf, accd.at[pl.ds(base_r + k * CH, CH)])
            pltpu.sync_copy(buf, accs.at[pl.ds(base_r + k * CH, CH)])
        _fill(1.0)
        plsc.subcore_barrier()

        ebase = wid * EPW

        def chunk(k, carry):
            off = ebase + k * CH
            pltpu.sync_copy(src_h.at[pl.ds(off, CH)], srcv)
            pltpu.sync_copy(dst_h.at[pl.ds(off, CH)], dstv)
            pltpu.sync_copy(buf, accd.at[dstv], add=True)
            pltpu.sync_copy(buf, accs.at[srcv], add=True)
            return carry
        lax.fori_loop(0, NCHUNK, chunk, 0)

        plsc.subcore_barrier()
        out_base = cid * NP + sid * ZROWS
        pltpu.sync_copy(accd.at[pl.ds(sid * ZROWS, ZROWS)],
                        degd_h.at[pl.ds(out_base, ZROWS)])
        pltpu.sync_copy(accs.at[pl.ds(sid * ZROWS, ZROWS)],
                        degs_h.at[pl.ds(out_base, ZROWS)])

    return sc_deg


_SC_CONV = _build_sc_conv()
_SC_DEG = _build_sc_deg()


def _ln64(x, g, b):
    m = jnp.mean(x, axis=-1, keepdims=True)
    v = jnp.mean((x - m) * (x - m), axis=-1, keepdims=True)
    return (x - m) * lax.rsqrt(v + EPS) * g + b


def _mm(a, b):
    return jnp.dot(a, b, preferred_element_type=jnp.float32)


def _ln2cols(ea):
    d = (ea[:, 0:1] - ea[:, 1:2]) * 0.5
    return d * lax.rsqrt(d * d + EPS)


def _embed_body(F, x_ref, lg_ref, lb_ref, w1_ref, b1_ref, w2_ref, b2_ref,
                o_ref):
    x = x_ref[...]
    mask = (lax.broadcasted_iota(jnp.int32, x.shape, 1) < F).astype(jnp.float32)
    m = jnp.sum(x * mask, axis=-1, keepdims=True) * (1.0 / F)
    v = jnp.sum(((x - m) * mask) ** 2, axis=-1, keepdims=True) * (1.0 / F)
    h = ((x - m) * lax.rsqrt(v + EPS) * lg_ref[...] + lb_ref[...]) * mask
    h = jnp.maximum(_mm(h, w1_ref[...]) + b1_ref[...], 0.0)
    o_ref[...] = jnp.maximum(_mm(h, w2_ref[...]) + b2_ref[...], 0.0)


def _embed(x_pad, ep, F, rows, grid):
    lg = jnp.pad(ep['ln_g'][None, :], ((0, 0), (0, 128 - F)))
    lb = jnp.pad(ep['ln_b'][None, :], ((0, 0), (0, 128 - F)))
    w1 = jnp.pad(ep['W1'].T, ((0, 128 - F), (0, 0)))
    return pl.pallas_call(
        functools.partial(_embed_body, F),
        grid=(grid,),
        in_specs=[
            pl.BlockSpec((rows, 128), lambda i: (i, 0)),
            pl.BlockSpec((1, 128), lambda i: (0, 0)),
            pl.BlockSpec((1, 128), lambda i: (0, 0)),
            pl.BlockSpec((128, EMB), lambda i: (0, 0)),
            pl.BlockSpec((1, EMB), lambda i: (0, 0)),
            pl.BlockSpec((EMB, EMB), lambda i: (0, 0)),
            pl.BlockSpec((1, EMB), lambda i: (0, 0)),
        ],
        out_specs=pl.BlockSpec((rows, EMB), lambda i: (i, 0)),
        out_shape=jax.ShapeDtypeStruct((rows * grid, EMB), jnp.float32),
    )(x_pad, lg, lb, w1, ep['b1'][None, :], ep['W2'].T, ep['b2'][None, :])


def _rowconv_body(x_ref, ea_ref, o_ref, wl_ref, bl_ref, we0_ref, we1_ref,
                  wr_ref, cg_ref, cb_ref, wf_ref, bf_ref, lg_ref, lb_ref,
                  w1a_ref, w1b_ref, b1_ref, w2_ref, b2_ref, out_ref):
    x = x_ref[...]
    n0 = _ln2cols(ea_ref[...])
    rvec = (_mm(o_ref[...], wr_ref[...]))[0:1]
    m = _mm(x, wl_ref[...]) + bl_ref[...] + n0 * (we0_ref[...] - we1_ref[...]) \
        + rvec
    m = jnp.maximum(_ln64(m, cg_ref[...], cb_ref[...]), 0.0)
    s = _mm(m, wf_ref[...]) + bf_ref[...]
    z = _ln64(s, lg_ref[...], lb_ref[...])
    h = jnp.maximum(_mm(x, w1a_ref[...]) + _mm(z, w1b_ref[...]) + b1_ref[...],
                    0.0)
    out_ref[...] = _mm(h, w2_ref[...]) + b2_ref[...]


def _reduceconv_body(x_ref, ea_ref, o_ref, wl_ref, bl_ref, we0_ref, we1_ref,
                     wr_ref, cg_ref, cb_ref, out_ref):
    i = pl.program_id(0)

    @pl.when(i == 0)
    def _():
        out_ref[...] = jnp.zeros_like(out_ref)

    x = x_ref[...]
    n0 = _ln2cols(ea_ref[...])
    lvec = (_mm(o_ref[...], wl_ref[...]) + bl_ref[...])[0:1]
    m = _mm(x, wr_ref[...]) + lvec + n0 * (we0_ref[...] - we1_ref[...])
    m = jnp.maximum(_ln64(m, cg_ref[...], cb_ref[...]), 0.0)
    rowid = i * BR + lax.broadcasted_iota(jnp.int32, (BR, 1), 0)
    m = m * (rowid < NN).astype(jnp.float32)
    out_ref[0:1, :] += jnp.sum(m, axis=0, keepdims=True)


def _objupdate_body(nreal, s_ref, o_ref, wf_ref, bf_ref, lg_ref, lb_ref,
                    w1a_ref, w1b_ref, b1_ref, w2_ref, b2_ref, out_ref):
    s = _mm(s_ref[...], wf_ref[...]) + nreal * bf_ref[...]
    z = _ln64(s, lg_ref[...], lb_ref[...])
    h = jnp.maximum(_mm(o_ref[...], w1a_ref[...]) + _mm(z, w1b_ref[...])
                    + b1_ref[...], 0.0)
    out_ref[...] = _mm(h, w2_ref[...]) + b2_ref[...]


def _ab_body(x_ref, y_ref, wl_ref, bl_ref, wr_ref, a_ref, b_ref):
    a_ref[...] = _mm(x_ref[...], wl_ref[...]) + bl_ref[...]
    b_ref[...] = _mm(y_ref[...], wr_ref[...])


def _rowconv_ab_body(x_ref, ea_ref, o_ref, wl_ref, bl_ref, we0_ref, we1_ref,
                     wr_ref, cg_ref, cb_ref, wf_ref, bf_ref, lg_ref, lb_ref,
                     w1a_ref, w1b_ref, b1_ref, w2_ref, b2_ref,
                     wl2_ref, bl2_ref, wr2_ref, y_ref,
                     out_ref, a_ref, b_ref):
    x = x_ref[...]
    n0 = _ln2cols(ea_ref[...])
    rvec = (_mm(o_ref[...], wr_ref[...]))[0:1]
    m = _mm(x, wl_ref[...]) + bl_ref[...] + n0 * (we0_ref[...] - we1_ref[...]) \
        + rvec
    m = jnp.maximum(_ln64(m, cg_ref[...], cb_ref[...]), 0.0)
    s = _mm(m, wf_ref[...]) + bf_ref[...]
    z = _ln64(s, lg_ref[...], lb_ref[...])
    h = jnp.maximum(_mm(x, w1a_ref[...]) + _mm(z, w1b_ref[...]) + b1_ref[...],
                    0.0)
    cn = _mm(h, w2_ref[...]) + b2_ref[...]
    out_ref[...] = cn
    a_ref[...] = _mm(cn, wl2_ref[...]) + bl2_ref[...]
    b_ref[...] = _mm(y_ref[...], wr2_ref[...])


def _rowconv_a_body(x_ref, ea_ref, o_ref, wl_ref, bl_ref, we0_ref, we1_ref,
                    wr_ref, cg_ref, cb_ref, wf_ref, bf_ref, lg_ref, lb_ref,
                    w1a_ref, w1b_ref, b1_ref, w2_ref, b2_ref,
                    wl2_ref, bl2_ref, out_ref, a_ref):
    x = x_ref[...]
    n0 = _ln2cols(ea_ref[...])
    rvec = (_mm(o_ref[...], wr_ref[...]))[0:1]
    m = _mm(x, wl_ref[...]) + bl_ref[...] + n0 * (we0_ref[...] - we1_ref[...]) \
        + rvec
    m = jnp.maximum(_ln64(m, cg_ref[...], cb_ref[...]), 0.0)
    s = _mm(m, wf_ref[...]) + bf_ref[...]
    z = _ln64(s, lg_ref[...], lb_ref[...])
    h = jnp.maximum(_mm(x, w1a_ref[...]) + _mm(z, w1b_ref[...]) + b1_ref[...],
                    0.0)
    cn = _mm(h, w2_ref[...]) + b2_ref[...]
    out_ref[...] = cn
    a_ref[...] = _mm(cn, wl2_ref[...]) + bl2_ref[...]


def _post_b_body(a0_ref, a1_ref, d0_ref, d1_ref, x_ref, wf_ref, bf_ref,
                 lg_ref, lb_ref, w1a_ref, w1b_ref, b1_ref, w2_ref, b2_ref,
                 wr2_ref, out_ref, b_ref):
    agg = a0_ref[...] + a1_ref[...]
    deg = d0_ref[:, 0:1] + d1_ref[:, 0:1]
    s = _mm(agg, wf_ref[...]) + deg * bf_ref[...]
    z = _ln64(s, lg_ref[...], lb_ref[...])
    h = jnp.maximum(_mm(x_ref[...], w1a_ref[...]) + _mm(z, w1b_ref[...])
                    + b1_ref[...], 0.0)
    cn = _mm(h, w2_ref[...]) + b2_ref[...]
    out_ref[...] = cn
    b_ref[...] = _mm(cn, wr2_ref[...])


def _post_body(a0_ref, a1_ref, d0_ref, d1_ref, x_ref, wf_ref, bf_ref,
               lg_ref, lb_ref, w1a_ref, w1b_ref, b1_ref, w2_ref, b2_ref,
               out_ref):
    agg = a0_ref[...] + a1_ref[...]
    deg = d0_ref[:, 0:1] + d1_ref[:, 0:1]
    s = _mm(agg, wf_ref[...]) + deg * bf_ref[...]
    z = _ln64(s, lg_ref[...], lb_ref[...])
    h = jnp.maximum(_mm(x_ref[...], w1a_ref[...]) + _mm(z, w1b_ref[...])
                    + b1_ref[...], 0.0)
    out_ref[...] = _mm(h, w2_ref[...]) + b2_ref[...]


def _head_body(x_ref, w1_ref, b1_ref, w2_ref, out_ref):
    h = jnp.maximum(_mm(x_ref[...], w1_ref[...]) + b1_ref[...], 0.0)
    out_ref[...] = jax.nn.sigmoid(_mm(h, w2_ref[...]))


def _wspec(r, c):
    return pl.BlockSpec((r, c), lambda i: (0, 0))


def _xspec(c):
    return pl.BlockSpec((BR, c), lambda i: (i, 0))


def _conv_we(cp, elg, elb):
    we = cp['We']
    we0 = (we[:, 0] * elg[0])[None, :]
    we1 = (we[:, 1] * elg[1])[None, :]
    ebias = we[:, 0] * elb[0] + we[:, 1] * elb[1]
    blv = (cp['bl'] + ebias)[None, :]
    return we0, we1, blv


def _rowconv(x, ea_pad, o8, cp, dp, lg, lb, we0, we1, blv):
    return pl.pallas_call(
        _rowconv_body,
        grid=(GRID,),
        in_specs=[
            _xspec(EMB), _xspec(128), _wspec(8, EMB),
            _wspec(EMB, EMB), _wspec(1, EMB), _wspec(1, EMB), _wspec(1, EMB),
            _wspec(EMB, EMB), _wspec(1, EMB), _wspec(1, EMB),
            _wspec(EMB, EMB), _wspec(1, EMB), _wspec(1, EMB), _wspec(1, EMB),
            _wspec(EMB, EMB), _wspec(EMB, EMB), _wspec(1, EMB),
            _wspec(EMB, EMB), _wspec(1, EMB),
        ],
        out_specs=_xspec(EMB),
        out_shape=jax.ShapeDtypeStruct((NP, EMB), jnp.float32),
    )(x, ea_pad, o8, cp['Wl'].T, blv, we0, we1, cp['Wr'].T,
      cp['g'][None, :], cp['bln'][None, :], cp['Wf'].T, cp['bf'][None, :],
      lg, lb, dp['W1'][:, :EMB].T, dp['W1'][:, EMB:].T, dp['b1'][None, :],
      dp['W2'].T, dp['b2'][None, :])


def _reduceconv(x, ea_pad, o8, cp, we0, we1, blv):
    return pl.pallas_call(
        _reduceconv_body,
        grid=(GRID,),
        in_specs=[
            _xspec(EMB), _xspec(128), _wspec(8, EMB),
            _wspec(EMB, EMB), _wspec(1, EMB), _wspec(1, EMB), _wspec(1, EMB),
            _wspec(EMB, EMB), _wspec(1, EMB), _wspec(1, EMB),
        ],
        out_specs=pl.BlockSpec((8, EMB), lambda i: (0, 0)),
        out_shape=jax.ShapeDtypeStruct((8, EMB), jnp.float32),
    )(x, ea_pad, o8, cp['Wl'].T, blv, we0, we1, cp['Wr'].T,
      cp['g'][None, :], cp['bln'][None, :])


def _objupdate(s8, o8, cp, dp, lg, lb, nreal):
    return pl.pallas_call(
        functools.partial(_objupdate_body, float(nreal)),
        grid=(1,),
        in_specs=[
            _wspec(8, EMB), _wspec(8, EMB),
            _wspec(EMB, EMB), _wspec(1, EMB), _wspec(1, EMB), _wspec(1, EMB),
            _wspec(EMB, EMB), _wspec(EMB, EMB), _wspec(1, EMB),
            _wspec(EMB, EMB), _wspec(1, EMB),
        ],
        out_specs=pl.BlockSpec((8, EMB), lambda i: (0, 0)),
        out_shape=jax.ShapeDtypeStruct((8, EMB), jnp.float32),
    )(s8, o8, cp['Wf'].T, cp['bf'][None, :], lg, lb,
      dp['W1'][:, :EMB].T, dp['W1'][:, EMB:].T, dp['b1'][None, :],
      dp['W2'].T, dp['b2'][None, :])


def _ab(x, y, cp, blv):
    return pl.pallas_call(
        _ab_body,
        grid=(GRID,),
        in_specs=[_xspec(EMB), _xspec(EMB),
                  _wspec(EMB, EMB), _wspec(1, EMB), _wspec(EMB, EMB)],
        out_specs=[_xspec(EMB), _xspec(EMB)],
        out_shape=[jax.ShapeDtypeStruct((NP, EMB), jnp.float32),
                   jax.ShapeDtypeStruct((NP, EMB), jnp.float32)],
    )(x, y, cp['Wl'].T, blv, cp['Wr'].T)


def _post(accp, degp, x, cp, dp, lg, lb):
    return pl.pallas_call(
        _post_body,
        grid=(GRID,),
        in_specs=[
            pl.BlockSpec((BR, EMB), lambda i: (i, 0)),
            pl.BlockSpec((BR, EMB), lambda i: (i + GRID, 0)),
            pl.BlockSpec((BR, 16), lambda i: (i, 0)),
            pl.BlockSpec((BR, 16), lambda i: (i + GRID, 0)),
            _xspec(EMB),
            _wspec(EMB, EMB), _wspec(1, EMB), _wspec(1, EMB), _wspec(1, EMB),
            _wspec(EMB, EMB), _wspec(EMB, EMB), _wspec(1, EMB),
            _wspec(EMB, EMB), _wspec(1, EMB),
        ],
        out_specs=_xspec(EMB),
        out_shape=jax.ShapeDtypeStruct((NP, EMB), jnp.float32),
    )(accp, accp, degp, degp, x, cp['Wf'].T, cp['bf'][None, :], lg, lb,
      dp['W1'][:, :EMB].T, dp['W1'][:, EMB:].T, dp['b1'][None, :],
      dp['W2'].T, dp['b2'][None, :])


def _rowconv_ab(x, ea_pad, o8, cp, dp, lg, lb, we0, we1, blv,
                cp2, blv2, y):
    return pl.pallas_call(
        _rowconv_ab_body,
        grid=(GRID,),
        in_specs=[
            _xspec(EMB), _xspec(128), _wspec(8, EMB),
            _wspec(EMB, EMB), _wspec(1, EMB), _wspec(1, EMB), _wspec(1, EMB),
            _wspec(EMB, EMB), _wspec(1, EMB), _wspec(1, EMB),
            _wspec(EMB, EMB), _wspec(1, EMB), _wspec(1, EMB), _wspec(1, EMB),
            _wspec(EMB, EMB), _wspec(EMB, EMB), _wspec(1, EMB),
            _wspec(EMB, EMB), _wspec(1, EMB),
            _wspec(EMB, EMB), _wspec(1, EMB), _wspec(EMB, EMB), _xspec(EMB),
        ],
        out_specs=[_xspec(EMB), _xspec(EMB), _xspec(EMB)],
        out_shape=[jax.ShapeDtypeStruct((NP, EMB), jnp.float32)] * 3,
    )(x, ea_pad, o8, cp['Wl'].T, blv, we0, we1, cp['Wr'].T,
      cp['g'][None, :], cp['bln'][None, :], cp['Wf'].T, cp['bf'][None, :],
      lg, lb, dp['W1'][:, :EMB].T, dp['W1'][:, EMB:].T, dp['b1'][None, :],
      dp['W2'].T, dp['b2'][None, :], cp2['Wl'].T, blv2, cp2['Wr'].T, y)


def _rowconv_a(x, ea_pad, o8, cp, dp, lg, lb, we0, we1, blv, cp2, blv2):
    return pl.pallas_call(
        _rowconv_a_body,
        grid=(GRID,),
        in_specs=[
            _xspec(EMB), _xspec(128), _wspec(8, EMB),
            _wspec(EMB, EMB), _wspec(1, EMB), _wspec(1, EMB), _wspec(1, EMB),
            _wspec(EMB, EMB), _wspec(1, EMB), _wspec(1, EMB),
            _wspec(EMB, EMB), _wspec(1, EMB), _wspec(1, EMB), _wspec(1, EMB),
            _wspec(EMB, EMB), _wspec(EMB, EMB), _wspec(1, EMB),
            _wspec(EMB, EMB), _wspec(1, EMB),
            _wspec(EMB, EMB), _wspec(1, EMB),
        ],
        out_specs=[_xspec(EMB), _xspec(EMB)],
        out_shape=[jax.ShapeDtypeStruct((NP, EMB), jnp.float32)] * 2,
    )(x, ea_pad, o8, cp['Wl'].T, blv, we0, we1, cp['Wr'].T,
      cp['g'][None, :], cp['bln'][None, :], cp['Wf'].T, cp['bf'][None, :],
      lg, lb, dp['W1'][:, :EMB].T, dp['W1'][:, EMB:].T, dp['b1'][None, :],
      dp['W2'].T, dp['b2'][None, :], cp2['Wl'].T, blv2)


def _post_b(accp, degp, x, cp, dp, lg, lb, cp2):
    return pl.pallas_call(
        _post_b_body,
        grid=(GRID,),
        in_specs=[
            pl.BlockSpec((BR, EMB), lambda i: (i, 0)),
            pl.BlockSpec((BR, EMB), lambda i: (i + GRID, 0)),
            pl.BlockSpec((BR, 16), lambda i: (i, 0)),
            pl.BlockSpec((BR, 16), lambda i: (i + GRID, 0)),
            _xspec(EMB),
            _wspec(EMB, EMB), _wspec(1, EMB), _wspec(1, EMB), _wspec(1, EMB),
            _wspec(EMB, EMB), _wspec(EMB, EMB), _wspec(1, EMB),
            _wspec(EMB, EMB), _wspec(1, EMB), _wspec(EMB, EMB),
        ],
        out_specs=[_xspec(EMB), _xspec(EMB)],
        out_shape=[jax.ShapeDtypeStruct((NP, EMB), jnp.float32)] * 2,
    )(accp, accp, degp, degp, x, cp['Wf'].T, cp['bf'][None, :], lg, lb,
      dp['W1'][:, :EMB].T, dp['W1'][:, EMB:].T, dp['b1'][None, :],
      dp['W2'].T, dp['b2'][None, :], cp2['Wr'].T)


def _head(x, w1t, b1, w2p):
    return pl.pallas_call(
        _head_body,
        grid=(GRID,),
        in_specs=[_xspec(EMB), _wspec(EMB, EMB), _wspec(1, EMB),
                  _wspec(EMB, 128)],
        out_specs=_xspec(128),
        out_shape=jax.ShapeDtypeStruct((NP, 128), jnp.float32),
    )(x, w1t, b1, w2p)


def kernel(x_u, x_c, x_o, ea_vc, ea_ov, ea_oc, ei_vc, ei_ov, ei_oc, params):
    p = params
    f32 = jnp.float32
    lg = p['ln_g'][None, :]
    lb = p['ln_b'][None, :]
    elg = p['edge_ln_g']
    elb = p['edge_ln_b']

    def padrc(a, rows, cols=128):
        return jnp.pad(a, ((0, rows - a.shape[0]), (0, cols - a.shape[1])))

    u = _embed(padrc(x_u, NP), p['ne0'], 14, BR, GRID)
    c = _embed(padrc(x_c, NP), p['ne1'], 6, BR, GRID)
    o = _embed(padrc(x_o, 8), p['ne2'], 2, 8, 1)

    ea_ov_pad = padrc(ea_ov, NP)
    ea_oc_pad = padrc(ea_oc, NP)

    vi = jnp.concatenate([ei_vc[0], jnp.full((EP - NE,), NN, jnp.int32)])
    ci = jnp.concatenate([ei_vc[1], jnp.full((EP - NE,), NN, jnp.int32)])
    eaf = jnp.concatenate([ea_vc, jnp.zeros((EP - NE, 2), f32)]).reshape(-1)

    degc, degv = _SC_DEG(vi, ci)

    for l in range(2):
        cp = p['conv%d_u_obj' % l]
        we0, we1, blv = _conv_we(cp, elg, elb)
        s8 = _reduceconv(u, ea_ov_pad, o, cp, we0, we1, blv)
        o = _objupdate(s8, o, cp, p['emb%d_obj' % l], lg, lb, NN)

        cpv = p['conv%d_u_con' % l]
        we0v, we1v, blvv = _conv_we(cpv, elg, elb)
        cp = p['conv%d_obj_con' % l]
        we0, we1, blv = _conv_we(cp, elg, elb)
        c, a_t, b_t = _rowconv_ab(c, ea_oc_pad, o, cp, p['emb%d_con' % l],
                                  lg, lb, we0, we1, blv, cpv, blvv, u)

        wconst = jnp.concatenate([we0v[0], we1v[0], cpv['g'], cpv['bln']])
        accp = _SC_CONV(a_t, b_t, vi, ci, eaf, wconst)
        cpu2 = p['conv%d_con_u' % l]
        we0u, we1u, blvu = _conv_we(cpu2, elg, elb)
        c, b2_t = _post_b(accp, degc, c, cpv, p['emb%d_con' % l], lg, lb,
                          cpu2)

        cp = p['conv%d_con_obj' % l]
        we0, we1, blv = _conv_we(cp, elg, elb)
        s8 = _reduceconv(c, ea_oc_pad, o, cp, we0, we1, blv)
        o = _objupdate(s8, o, cp, p['emb%d_obj' % l], lg, lb, NN)

        cp = p['conv%d_obj_u' % l]
        we0, we1, blv = _conv_we(cp, elg, elb)
        u, a2_t = _rowconv_a(u, ea_ov_pad, o, cp, p['emb%d_u' % l], lg, lb,
                             we0, we1, blv, cpu2, blvu)

        wconst = jnp.concatenate([we0u[0], we1u[0], cpu2['g'], cpu2['bln']])
        accp = _SC_CONV(a2_t, b2_t, ci, vi, eaf, wconst)
        u = _post(accp, degv, u, cpu2, p['emb%d_u' % l], lg, lb)

    w2p = jnp.pad(p['out_W2'].T, ((0, 0), (0, 127)))
    res = _head(u, p['out_W1'].T, p['out_b1'][None, :], w2p)
    return res[:NN, :1]
